# baseline (speedup 1.0000x reference)
import functools

import jax
import jax.numpy as jnp
from jax import lax
from jax.experimental import pallas as pl
from jax.experimental.pallas import tpu as pltpu
from jax.experimental.pallas import tpu_sc as plsc

B = 16384
MF_D = 16
MLP_D = 10
NC = 2
NS = 16
NW = NC * NS
BPW = B // NW


@functools.cache
def _make_sc_gather():
    mesh = plsc.VectorSubcoreMesh(core_axis_name="c", subcore_axis_name="s")

    @functools.partial(
        pl.kernel,
        mesh=mesh,
        compiler_params=pltpu.CompilerParams(use_tc_tiling_on_sc=True),
        out_type=[
            jax.ShapeDtypeStruct((B, MF_D), jnp.float32),
            jax.ShapeDtypeStruct((B, MF_D), jnp.float32),
            jax.ShapeDtypeStruct((B, MLP_D), jnp.float32),
            jax.ShapeDtypeStruct((B, MLP_D), jnp.float32),
        ],
        scratch_types=[
            pltpu.VMEM((BPW,), jnp.int32),
            pltpu.VMEM((BPW,), jnp.int32),
            pltpu.SemaphoreType.DMA,
        ],
    )
    def _sc_gather(u_hbm, i_hbm, mfu_hbm, mfi_hbm, mlpu_hbm, mlpi_hbm,
                   o_mfu, o_mfi, o_mlpu, o_mlpi,
                   uv, iv, sem):
        wid = lax.axis_index("s") * NC + lax.axis_index("c")
        base = wid * BPW
        pltpu.sync_copy(u_hbm.at[pl.ds(base, BPW)], uv)
        pltpu.sync_copy(i_hbm.at[pl.ds(base, BPW)], iv)

        def body(g, _):
            r0 = g * 16
            uvec = uv[pl.ds(r0, 16)]
            ivec = iv[pl.ds(r0, 16)]
            for j in range(16):
                r = base + r0 + j
                a = uvec[j]
                b = ivec[j]
                pltpu.async_copy(mfu_hbm.at[pl.ds(a, 1)],
                                 o_mfu.at[pl.ds(r, 1)], sem)
                pltpu.async_copy(mfi_hbm.at[pl.ds(b, 1)],
                                 o_mfi.at[pl.ds(r, 1)], sem)
                pltpu.async_copy(mlpu_hbm.at[pl.ds(a, 1)],
                                 o_mlpu.at[pl.ds(r, 1)], sem)
                pltpu.async_copy(mlpi_hbm.at[pl.ds(b, 1)],
                                 o_mlpi.at[pl.ds(r, 1)], sem)
            return _

        lax.fori_loop(0, BPW // 16, body, 0)
        pltpu.make_async_copy(mfu_hbm.at[pl.ds(0, BPW)],
                              o_mfu.at[pl.ds(base, BPW)], sem).wait()
        pltpu.make_async_copy(mfi_hbm.at[pl.ds(0, BPW)],
                              o_mfi.at[pl.ds(base, BPW)], sem).wait()
        pltpu.make_async_copy(mlpu_hbm.at[pl.ds(0, BPW)],
                              o_mlpu.at[pl.ds(base, BPW)], sem).wait()
        pltpu.make_async_copy(mlpi_hbm.at[pl.ds(0, BPW)],
                              o_mlpi.at[pl.ds(base, BPW)], sem).wait()

    return _sc_gather


BM = 2048


def _tc_mlp_body(mfu, mfi, mlpu, mlpi, W1a, W1b, b1, W2, b2, W3, b3,
                 Wl, bl, w2la, w2lb, b2l, out):
    f32 = jnp.float32
    x = (jnp.dot(mlpu[...], W1a[...], preferred_element_type=f32)
         + jnp.dot(mlpi[...], W1b[...], preferred_element_type=f32)
         + b1[...])
    x = jnp.maximum(x, 0.0)
    x = jnp.dot(x, W2[...], preferred_element_type=f32) + b2[...]
    x = jnp.maximum(x, 0.0)
    x = jnp.dot(x, W3[...], preferred_element_type=f32) + b3[...]
    x = jnp.maximum(x, 0.0)
    mlp_vec = jnp.dot(x, Wl[...], preferred_element_type=f32) + bl[...]
    mf_vec = mfu[...] * mfi[...]
    logit = (jnp.dot(mf_vec, w2la[...], preferred_element_type=f32)
             + jnp.dot(mlp_vec, w2lb[...], preferred_element_type=f32)
             + b2l[...])
    out[...] = jax.nn.sigmoid(logit)


def _tc_mlp(mfu, mfi, mlpu, mlpi, W1a, W1b, b1, W2, b2, W3, b3,
            Wl, bl, w2la, w2lb, b2l):
    def row_block(d):
        return pl.BlockSpec((BM, d), lambda m: (m, 0))

    def full(a):
        return pl.BlockSpec(a.shape, lambda m: (0,) * a.ndim)

    return pl.pallas_call(
        _tc_mlp_body,
        grid=(B // BM,),
        in_specs=[
            row_block(MF_D), row_block(MF_D), row_block(MLP_D),
            row_block(MLP_D),
            full(W1a), full(W1b), full(b1), full(W2), full(b2),
            full(W3), full(b3), full(Wl), full(bl),
            full(w2la), full(w2lb), full(b2l),
        ],
        out_specs=pl.BlockSpec((BM, 1), lambda m: (m, 0)),
        out_shape=jax.ShapeDtypeStruct((B, 1), jnp.float32),
    )(mfu, mfi, mlpu, mlpi, W1a, W1b, b1, W2, b2, W3, b3,
      Wl, bl, w2la, w2lb, b2l)


def kernel(inputs, mf_user, mf_item, mlp_user, mlp_item,
           W1, b1, W2, b2, W3, b3, Wl, bl, W2l, b2l):
    u = inputs[:, 0]
    i = inputs[:, 1]
    mfu, mfi, mlpu, mlpi = _make_sc_gather()(
        u, i, mf_user, mf_item, mlp_user, mlp_item)
    return _tc_mlp(
        mfu, mfi, mlpu, mlpi,
        W1[:MLP_D], W1[MLP_D:], b1.reshape(1, -1),
        W2, b2.reshape(1, -1), W3, b3.reshape(1, -1),
        Wl, bl.reshape(1, -1),
        W2l[:MF_D], W2l[MF_D:], b2l.reshape(1, 1),
    )


# speedup vs baseline: 1.0563x; 1.0563x over previous
import functools

import jax
import jax.numpy as jnp
from jax import lax
from jax.experimental import pallas as pl
from jax.experimental.pallas import tpu as pltpu
from jax.experimental.pallas import tpu_sc as plsc

B = 16384
NT = 1000000
MF_D = 16
MLP_D = 10
NC = 2
NS = 16
NW = NC * NS
BPW = B // NW
BR = 2000
CH = 256

C_MFU = 0
C_MFI = 16
C_MLU = 32
C_MLI = 42


def _repack_body(mfu, mfi, mlpu, mlpi, out):
    pad = jnp.zeros((BR, 128 - 2 * MF_D - 2 * MLP_D), jnp.float32)
    out[...] = jnp.concatenate(
        [mfu[...], mfi[...], mlpu[...], mlpi[...], pad], axis=1)


def _repack(mf_user, mf_item, mlp_user, mlp_item):
    return pl.pallas_call(
        _repack_body,
        grid=(NT // BR,),
        in_specs=[
            pl.BlockSpec((BR, MF_D), lambda m: (m, 0)),
            pl.BlockSpec((BR, MF_D), lambda m: (m, 0)),
            pl.BlockSpec((BR, MLP_D), lambda m: (m, 0)),
            pl.BlockSpec((BR, MLP_D), lambda m: (m, 0)),
        ],
        out_specs=pl.BlockSpec((BR, 128), lambda m: (m, 0)),
        out_shape=jax.ShapeDtypeStruct((NT, 128), jnp.float32),
        compiler_params=pltpu.CompilerParams(
            dimension_semantics=("arbitrary",)),
    )(mf_user, mf_item, mlp_user, mlp_item)


@functools.cache
def _make_sc_gather():
    mesh = plsc.VectorSubcoreMesh(core_axis_name="c", subcore_axis_name="s")

    @functools.partial(
        pl.kernel,
        mesh=mesh,
        compiler_params=pltpu.CompilerParams(use_tc_tiling_on_sc=True),
        out_type=[
            jax.ShapeDtypeStruct((B, 128), jnp.float32),
            jax.ShapeDtypeStruct((B, 128), jnp.float32),
        ],
        scratch_types=[
            pltpu.VMEM((BPW,), jnp.int32),
            pltpu.VMEM((BPW,), jnp.int32),
            pltpu.VMEM((CH, 128), jnp.float32),
            pltpu.VMEM((CH, 128), jnp.float32),
            pltpu.SemaphoreType.DMA,
        ],
    )
    def _sc_gather(u_hbm, i_hbm, fused_hbm,
                   o_u, o_i, uv, iv, bu, bi, sem):
        wid = lax.axis_index("s") * NC + lax.axis_index("c")
        base = wid * BPW
        pltpu.sync_copy(u_hbm.at[pl.ds(base, BPW)], uv)
        pltpu.sync_copy(i_hbm.at[pl.ds(base, BPW)], iv)
        for c in range(BPW // CH):
            cu = pltpu.async_copy(fused_hbm.at[uv.at[pl.ds(c * CH, CH)]],
                                  bu, sem)
            ci = pltpu.async_copy(fused_hbm.at[iv.at[pl.ds(c * CH, CH)]],
                                  bi, sem)
            cu.wait()
            ci.wait()
            pltpu.sync_copy(bu, o_u.at[pl.ds(base + c * CH, CH)])
            pltpu.sync_copy(bi, o_i.at[pl.ds(base + c * CH, CH)])

    return _sc_gather


BM = 2048


def _tc_mlp_body(gu, gi, W1a, W1b, b1, W2, b2, W3, b3,
                 Wl, bl, w2la, w2lb, b2l, out):
    f32 = jnp.float32
    gu_ = gu[...]
    gi_ = gi[...]
    mfu = gu_[:, C_MFU:C_MFU + MF_D]
    mlpu = gu_[:, C_MLU:C_MLU + MLP_D]
    mfi = gi_[:, C_MFI:C_MFI + MF_D]
    mlpi = gi_[:, C_MLI:C_MLI + MLP_D]
    x = (jnp.dot(mlpu, W1a[...], preferred_element_type=f32)
         + jnp.dot(mlpi, W1b[...], preferred_element_type=f32)
         + b1[...])
    x = jnp.maximum(x, 0.0)
    x = jnp.dot(x, W2[...], preferred_element_type=f32) + b2[...]
    x = jnp.maximum(x, 0.0)
    x = jnp.dot(x, W3[...], preferred_element_type=f32) + b3[...]
    x = jnp.maximum(x, 0.0)
    mlp_vec = jnp.dot(x, Wl[...], preferred_element_type=f32) + bl[...]
    mf_vec = mfu * mfi
    logit = (jnp.dot(mf_vec, w2la[...], preferred_element_type=f32)
             + jnp.dot(mlp_vec, w2lb[...], preferred_element_type=f32)
             + b2l[...])
    out[...] = jax.nn.sigmoid(logit)


def _tc_mlp(gu, gi, W1a, W1b, b1, W2, b2, W3, b3, Wl, bl, w2la, w2lb, b2l):
    def full(a):
        return pl.BlockSpec(a.shape, lambda m: (0,) * a.ndim)

    return pl.pallas_call(
        _tc_mlp_body,
        grid=(B // BM,),
        in_specs=[
            pl.BlockSpec((BM, 128), lambda m: (m, 0)),
            pl.BlockSpec((BM, 128), lambda m: (m, 0)),
            full(W1a), full(W1b), full(b1), full(W2), full(b2),
            full(W3), full(b3), full(Wl), full(bl),
            full(w2la), full(w2lb), full(b2l),
        ],
        out_specs=pl.BlockSpec((BM, 1), lambda m: (m, 0)),
        out_shape=jax.ShapeDtypeStruct((B, 1), jnp.float32),
    )(gu, gi, W1a, W1b, b1, W2, b2, W3, b3, Wl, bl, w2la, w2lb, b2l)


def kernel(inputs, mf_user, mf_item, mlp_user, mlp_item,
           W1, b1, W2, b2, W3, b3, Wl, bl, W2l, b2l):
    u = inputs[:, 0]
    i = inputs[:, 1]
    fused = _repack(mf_user, mf_item, mlp_user, mlp_item)
    gu, gi = _make_sc_gather()(u, i, fused)
    return _tc_mlp(
        gu, gi,
        W1[:MLP_D], W1[MLP_D:], b1.reshape(1, -1),
        W2, b2.reshape(1, -1), W3, b3.reshape(1, -1),
        Wl, bl.reshape(1, -1),
        W2l[:MF_D], W2l[MF_D:], b2l.reshape(1, 1),
    )


# speedup vs baseline: 1.0745x; 1.0172x over previous
import functools

import jax
import jax.numpy as jnp
from jax import lax
from jax.experimental import pallas as pl
from jax.experimental.pallas import tpu as pltpu
from jax.experimental.pallas import tpu_sc as plsc

B = 16384
NT = 1000000
MF_D = 16
MLP_D = 10
NC = 2
NS = 16
NW = NC * NS
BPW = B // NW
BR = 2000
CH = 256

C_MFU = 0
C_MFI = 16
C_MLU = 32
C_MLI = 42


def _repack_body(mfu, mfi, mlpu, mlpi, p0, p1, p2, p3, out):
    f32 = jnp.float32
    out[...] = (
        jnp.dot(mfu[...], p0[...], preferred_element_type=f32)
        + jnp.dot(mfi[...], p1[...], preferred_element_type=f32)
        + jnp.dot(mlpu[...], p2[...], preferred_element_type=f32)
        + jnp.dot(mlpi[...], p3[...], preferred_element_type=f32))


def _repack(mf_user, mf_item, mlp_user, mlp_item):
    p0 = jnp.eye(MF_D, 128, C_MFU, jnp.float32)
    p1 = jnp.eye(MF_D, 128, C_MFI, jnp.float32)
    p2 = jnp.eye(MLP_D, 128, C_MLU, jnp.float32)
    p3 = jnp.eye(MLP_D, 128, C_MLI, jnp.float32)

    def full(a):
        return pl.BlockSpec(a.shape, lambda m: (0, 0))

    return pl.pallas_call(
        _repack_body,
        grid=(NT // BR,),
        in_specs=[
            pl.BlockSpec((BR, MF_D), lambda m: (m, 0)),
            pl.BlockSpec((BR, MF_D), lambda m: (m, 0)),
            pl.BlockSpec((BR, MLP_D), lambda m: (m, 0)),
            pl.BlockSpec((BR, MLP_D), lambda m: (m, 0)),
            full(p0), full(p1), full(p2), full(p3),
        ],
        out_specs=pl.BlockSpec((BR, 128), lambda m: (m, 0)),
        out_shape=jax.ShapeDtypeStruct((NT, 128), jnp.float32),
        compiler_params=pltpu.CompilerParams(
            dimension_semantics=("arbitrary",)),
    )(mf_user, mf_item, mlp_user, mlp_item, p0, p1, p2, p3)


@functools.cache
def _make_sc_gather():
    mesh = plsc.VectorSubcoreMesh(core_axis_name="c", subcore_axis_name="s")

    @functools.partial(
        pl.kernel,
        mesh=mesh,
        compiler_params=pltpu.CompilerParams(use_tc_tiling_on_sc=True),
        out_type=[
            jax.ShapeDtypeStruct((B, 128), jnp.float32),
            jax.ShapeDtypeStruct((B, 128), jnp.float32),
        ],
        scratch_types=[
            pltpu.VMEM((BPW,), jnp.int32),
            pltpu.VMEM((BPW,), jnp.int32),
            pltpu.VMEM((CH, 128), jnp.float32),
            pltpu.VMEM((CH, 128), jnp.float32),
            pltpu.SemaphoreType.DMA,
        ],
    )
    def _sc_gather(u_hbm, i_hbm, fused_hbm,
                   o_u, o_i, uv, iv, bu, bi, sem):
        wid = lax.axis_index("s") * NC + lax.axis_index("c")
        base = wid * BPW
        pltpu.sync_copy(u_hbm.at[pl.ds(base, BPW)], uv)
        pltpu.sync_copy(i_hbm.at[pl.ds(base, BPW)], iv)
        for c in range(BPW // CH):
            cu = pltpu.async_copy(fused_hbm.at[uv.at[pl.ds(c * CH, CH)]],
                                  bu, sem)
            ci = pltpu.async_copy(fused_hbm.at[iv.at[pl.ds(c * CH, CH)]],
                                  bi, sem)
            cu.wait()
            ci.wait()
            pltpu.sync_copy(bu, o_u.at[pl.ds(base + c * CH, CH)])
            pltpu.sync_copy(bi, o_i.at[pl.ds(base + c * CH, CH)])

    return _sc_gather


BM = 2048


def _tc_mlp_body(gu, gi, W1a, W1b, b1, W2, b2, W3, b3,
                 Wl, bl, w2la, w2lb, b2l, out):
    f32 = jnp.float32
    gu_ = gu[...]
    gi_ = gi[...]
    mfu = gu_[:, C_MFU:C_MFU + MF_D]
    mlpu = gu_[:, C_MLU:C_MLU + MLP_D]
    mfi = gi_[:, C_MFI:C_MFI + MF_D]
    mlpi = gi_[:, C_MLI:C_MLI + MLP_D]
    x = (jnp.dot(mlpu, W1a[...], preferred_element_type=f32)
         + jnp.dot(mlpi, W1b[...], preferred_element_type=f32)
         + b1[...])
    x = jnp.maximum(x, 0.0)
    x = jnp.dot(x, W2[...], preferred_element_type=f32) + b2[...]
    x = jnp.maximum(x, 0.0)
    x = jnp.dot(x, W3[...], preferred_element_type=f32) + b3[...]
    x = jnp.maximum(x, 0.0)
    mlp_vec = jnp.dot(x, Wl[...], preferred_element_type=f32) + bl[...]
    mf_vec = mfu * mfi
    logit = (jnp.dot(mf_vec, w2la[...], preferred_element_type=f32)
             + jnp.dot(mlp_vec, w2lb[...], preferred_element_type=f32)
             + b2l[...])
    out[...] = jax.nn.sigmoid(logit)


def _tc_mlp(gu, gi, W1a, W1b, b1, W2, b2, W3, b3, Wl, bl, w2la, w2lb, b2l):
    def full(a):
        return pl.BlockSpec(a.shape, lambda m: (0,) * a.ndim)

    return pl.pallas_call(
        _tc_mlp_body,
        grid=(B // BM,),
        in_specs=[
            pl.BlockSpec((BM, 128), lambda m: (m, 0)),
            pl.BlockSpec((BM, 128), lambda m: (m, 0)),
            full(W1a), full(W1b), full(b1), full(W2), full(b2),
            full(W3), full(b3), full(Wl), full(bl),
            full(w2la), full(w2lb), full(b2l),
        ],
        out_specs=pl.BlockSpec((BM, 1), lambda m: (m, 0)),
        out_shape=jax.ShapeDtypeStruct((B, 1), jnp.float32),
    )(gu, gi, W1a, W1b, b1, W2, b2, W3, b3, Wl, bl, w2la, w2lb, b2l)


def kernel(inputs, mf_user, mf_item, mlp_user, mlp_item,
           W1, b1, W2, b2, W3, b3, Wl, bl, W2l, b2l):
    u = inputs[:, 0]
    i = inputs[:, 1]
    fused = _repack(mf_user, mf_item, mlp_user, mlp_item)
    gu, gi = _make_sc_gather()(u, i, fused)
    return _tc_mlp(
        gu, gi,
        W1[:MLP_D], W1[MLP_D:], b1.reshape(1, -1),
        W2, b2.reshape(1, -1), W3, b3.reshape(1, -1),
        Wl, bl.reshape(1, -1),
        W2l[:MF_D], W2l[MF_D:], b2l.reshape(1, 1),
    )


# speedup vs baseline: 1.9078x; 1.7756x over previous
import functools

import jax
import jax.numpy as jnp
from jax import lax
from jax.experimental import pallas as pl
from jax.experimental.pallas import tpu as pltpu
from jax.experimental.pallas import tpu_sc as plsc

B = 16384
MF_D = 16
MLP_D = 10
NC = 2
NS = 16
NW = NC * NS
BPW = B // NW
RR = 128


@functools.cache
def _make_sc_gather():
    mesh = plsc.VectorSubcoreMesh(core_axis_name="c", subcore_axis_name="s")

    @functools.partial(
        pl.kernel,
        mesh=mesh,
        compiler_params=pltpu.CompilerParams(use_tc_tiling_on_sc=True),
        out_type=[
            jax.ShapeDtypeStruct((B, MF_D), jnp.float32),
            jax.ShapeDtypeStruct((B, MF_D), jnp.float32),
            jax.ShapeDtypeStruct((B, MLP_D), jnp.float32),
            jax.ShapeDtypeStruct((B, MLP_D), jnp.float32),
        ],
        scratch_types=[
            pltpu.VMEM((BPW,), jnp.int32),
            pltpu.VMEM((BPW,), jnp.int32),
            pltpu.VMEM((RR, MF_D), jnp.float32),
            pltpu.VMEM((RR, MF_D), jnp.float32),
            pltpu.VMEM((RR, MLP_D), jnp.float32),
            pltpu.VMEM((RR, MLP_D), jnp.float32),
            pltpu.SemaphoreType.DMA,
        ],
    )
    def _sc_gather(u_hbm, i_hbm, mfu_hbm, mfi_hbm, mlpu_hbm, mlpi_hbm,
                   o_mfu, o_mfi, o_mlpu, o_mlpi,
                   uv, iv, bmfu, bmfi, bmlpu, bmlpi, sem):
        wid = lax.axis_index("s") * NC + lax.axis_index("c")
        base = wid * BPW
        pltpu.sync_copy(u_hbm.at[pl.ds(base, BPW)], uv)
        pltpu.sync_copy(i_hbm.at[pl.ds(base, BPW)], iv)

        def round_(t, _):
            r0 = t * RR

            def body(g, _):
                gr = r0 + g * 16
                uvec = uv[pl.ds(gr, 16)]
                ivec = iv[pl.ds(gr, 16)]
                for j in range(16):
                    r = g * 16 + j
                    a = uvec[j]
                    b = ivec[j]
                    pltpu.async_copy(mfu_hbm.at[pl.ds(a, 1)],
                                     bmfu.at[pl.ds(r, 1)], sem)
                    pltpu.async_copy(mfi_hbm.at[pl.ds(b, 1)],
                                     bmfi.at[pl.ds(r, 1)], sem)
                    pltpu.async_copy(mlpu_hbm.at[pl.ds(a, 1)],
                                     bmlpu.at[pl.ds(r, 1)], sem)
                    pltpu.async_copy(mlpi_hbm.at[pl.ds(b, 1)],
                                     bmlpi.at[pl.ds(r, 1)], sem)
                return _

            lax.fori_loop(0, RR // 16, body, 0)
            ob = base + r0
            pltpu.make_async_copy(o_mfu.at[pl.ds(ob, RR)], bmfu, sem).wait()
            pltpu.make_async_copy(o_mfi.at[pl.ds(ob, RR)], bmfi, sem).wait()
            pltpu.make_async_copy(o_mlpu.at[pl.ds(ob, RR)], bmlpu, sem).wait()
            pltpu.make_async_copy(o_mlpi.at[pl.ds(ob, RR)], bmlpi, sem).wait()
            pltpu.sync_copy(bmfu, o_mfu.at[pl.ds(ob, RR)])
            pltpu.sync_copy(bmfi, o_mfi.at[pl.ds(ob, RR)])
            pltpu.sync_copy(bmlpu, o_mlpu.at[pl.ds(ob, RR)])
            pltpu.sync_copy(bmlpi, o_mlpi.at[pl.ds(ob, RR)])
            return _

        lax.fori_loop(0, BPW // RR, round_, 0)

    return _sc_gather


BM = 2048


def _tc_mlp_body(mfu, mfi, mlpu, mlpi, W1a, W1b, b1, W2, b2, W3, b3,
                 Wl, bl, w2la, w2lb, b2l, out):
    f32 = jnp.float32
    x = (jnp.dot(mlpu[...], W1a[...], preferred_element_type=f32)
         + jnp.dot(mlpi[...], W1b[...], preferred_element_type=f32)
         + b1[...])
    x = jnp.maximum(x, 0.0)
    x = jnp.dot(x, W2[...], preferred_element_type=f32) + b2[...]
    x = jnp.maximum(x, 0.0)
    x = jnp.dot(x, W3[...], preferred_element_type=f32) + b3[...]
    x = jnp.maximum(x, 0.0)
    mlp_vec = jnp.dot(x, Wl[...], preferred_element_type=f32) + bl[...]
    mf_vec = mfu[...] * mfi[...]
    logit = (jnp.dot(mf_vec, w2la[...], preferred_element_type=f32)
             + jnp.dot(mlp_vec, w2lb[...], preferred_element_type=f32)
             + b2l[...])
    out[...] = jax.nn.sigmoid(logit)


def _tc_mlp(mfu, mfi, mlpu, mlpi, W1a, W1b, b1, W2, b2, W3, b3,
            Wl, bl, w2la, w2lb, b2l):
    def row_block(d):
        return pl.BlockSpec((BM, d), lambda m: (m, 0))

    def full(a):
        return pl.BlockSpec(a.shape, lambda m: (0,) * a.ndim)

    return pl.pallas_call(
        _tc_mlp_body,
        grid=(B // BM,),
        in_specs=[
            row_block(MF_D), row_block(MF_D), row_block(MLP_D),
            row_block(MLP_D),
            full(W1a), full(W1b), full(b1), full(W2), full(b2),
            full(W3), full(b3), full(Wl), full(bl),
            full(w2la), full(w2lb), full(b2l),
        ],
        out_specs=pl.BlockSpec((BM, 1), lambda m: (m, 0)),
        out_shape=jax.ShapeDtypeStruct((B, 1), jnp.float32),
    )(mfu, mfi, mlpu, mlpi, W1a, W1b, b1, W2, b2, W3, b3,
      Wl, bl, w2la, w2lb, b2l)


def kernel(inputs, mf_user, mf_item, mlp_user, mlp_item,
           W1, b1, W2, b2, W3, b3, Wl, bl, W2l, b2l):
    u = inputs[:, 0]
    i = inputs[:, 1]
    mfu, mfi, mlpu, mlpi = _make_sc_gather()(
        u, i, mf_user, mf_item, mlp_user, mlp_item)
    return _tc_mlp(
        mfu, mfi, mlpu, mlpi,
        W1[:MLP_D], W1[MLP_D:], b1.reshape(1, -1),
        W2, b2.reshape(1, -1), W3, b3.reshape(1, -1),
        Wl, bl.reshape(1, -1),
        W2l[:MF_D], W2l[MF_D:], b2l.reshape(1, 1),
    )
